# Initial kernel scaffold; baseline (speedup 1.0000x reference)
import functools

import jax
import jax.numpy as jnp
from jax import lax
from jax.experimental import pallas as pl
from jax.experimental.pallas import tpu as pltpu
from jax.experimental.pallas import tpu_sc as plsc

N = 100000
E = 3200000
NPAD = 100096
ROWS = NPAD // 128
NC = 2
NS = 16
NW = NC * NS
K = 8
RPW = 784
NBURST = RPW // K
EPAD = NW * RPW * 128
NSLICE = NPAD // NS


def _agg_body(tab_hbm, src_hbm, dst_hbm, out_hbm,
              tab_s, acc_s, src_v, dst_v, msg_v, gsem, ssem):
    cid = lax.axis_index("c")
    sid = lax.axis_index("s")
    wid = cid * NS + sid
    r0 = sid * NSLICE
    pltpu.sync_copy(tab_hbm.at[pl.ds(r0, NSLICE)], tab_s.at[pl.ds(r0, NSLICE)])
    pltpu.sync_copy(tab_hbm.at[pl.ds(r0, NSLICE)], acc_s.at[pl.ds(r0, NSLICE)])
    plsc.subcore_barrier()

    def burst(j, carry):
        base = j * K
        pltpu.sync_copy(src_hbm.at[wid, pl.ds(base, K)], src_v)
        pltpu.sync_copy(dst_hbm.at[wid, pl.ds(base, K)], dst_v)
        gds = [pltpu.async_copy(tab_s.at[src_v.at[b]], msg_v.at[b], gsem)
               for b in range(K)]
        for d in gds:
            d.wait()
        sds = [pltpu.async_copy(msg_v.at[b], acc_s.at[dst_v.at[b]], ssem,
                                add=True)
               for b in range(K)]
        for d in sds:
            d.wait()
        return carry

    lax.fori_loop(0, NBURST, burst, 0)
    plsc.subcore_barrier()
    pltpu.sync_copy(acc_s.at[pl.ds(r0, NSLICE)],
                    out_hbm.at[cid, pl.ds(r0, NSLICE)])


_agg = functools.partial(
    pl.kernel,
    out_type=jax.ShapeDtypeStruct((NC, NPAD, 2), jnp.float32),
    mesh=plsc.VectorSubcoreMesh(core_axis_name="c", subcore_axis_name="s",
                                num_cores=NC, num_subcores=NS),
    scratch_types=[
        pltpu.VMEM_SHARED((NPAD, 2), jnp.float32),
        pltpu.VMEM_SHARED((NPAD, 2), jnp.float32),
        pltpu.VMEM((K, 128), jnp.int32),
        pltpu.VMEM((K, 128), jnp.int32),
        pltpu.VMEM((K, 128, 2), jnp.float32),
        pltpu.SemaphoreType.DMA,
        pltpu.SemaphoreType.DMA,
    ],
)(_agg_body)


def _mlp1_body(x0, x1, a00, a01, a10, a11, w1a, b1a, w1b, b1b, w2a, g0, g1):
    z0 = a00[...] + a10[...] - x0[...]
    z1 = a01[...] + a11[...] - x1[...]
    u0 = jnp.maximum(z0 * w1a[0, 0] + z1 * w1a[1, 0] + b1a[0], 0.0)
    u1 = jnp.maximum(z0 * w1a[0, 1] + z1 * w1a[1, 1] + b1a[1], 0.0)
    acc0 = jnp.zeros_like(z0)
    acc1 = jnp.zeros_like(z0)
    for k in range(16):
        h = jnp.maximum(u0 * w1b[0, k] + u1 * w1b[1, k] + b1b[k], 0.0)
        acc0 = acc0 + h * w2a[k, 0]
        acc1 = acc1 + h * w2a[k, 1]
    g0[...] = acc0
    g1[...] = acc1


_VS = pl.BlockSpec(memory_space=pltpu.VMEM)
_SS = pl.BlockSpec(memory_space=pltpu.SMEM)

_mlp1 = pl.pallas_call(
    _mlp1_body,
    out_shape=[jax.ShapeDtypeStruct((ROWS, 128), jnp.float32)] * 2,
    in_specs=[_VS] * 6 + [_SS] * 5,
    out_specs=[_VS] * 2,
)


def _mlp2_body(g0, g1, c00, c01, c10, c11, b2a, w2b, b2b, o0, o1):
    s0 = jnp.maximum(c00[...] + c10[...] - g0[...] + b2a[0], 0.0)
    s1 = jnp.maximum(c01[...] + c11[...] - g1[...] + b2a[1], 0.0)
    h0 = s0 * w2b[0, 0] + s1 * w2b[1, 0] + b2b[0]
    h1 = s0 * w2b[0, 1] + s1 * w2b[1, 1] + b2b[1]
    m = jnp.maximum(h0, h1)
    lse = m + jnp.log(jnp.exp(h0 - m) + jnp.exp(h1 - m))
    o0[...] = h0 - lse
    o1[...] = h1 - lse


_mlp2 = pl.pallas_call(
    _mlp2_body,
    out_shape=[jax.ShapeDtypeStruct((ROWS, 128), jnp.float32)] * 2,
    in_specs=[_VS] * 6 + [_SS] * 3,
    out_specs=[_VS] * 2,
)


def kernel(x, edge_index, W1a, b1a, W1b, b1b, W2a, b2a, W2b, b2b):
    src = edge_index[0]
    dst = edge_index[1]
    padn = EPAD - E
    pad_idx = jnp.full((padn,), N, jnp.int32)
    srcp = jnp.concatenate([src, pad_idx]).reshape(NW, RPW, 128)
    dstp = jnp.concatenate([dst, pad_idx]).reshape(NW, RPW, 128)
    xpad = jnp.pad(x, ((0, NPAD - N), (0, 0)))

    p1 = _agg(xpad, srcp, dstp)
    x0 = xpad[:, 0].reshape(ROWS, 128)
    x1 = xpad[:, 1].reshape(ROWS, 128)
    g0, g1 = _mlp1(x0, x1,
                   p1[0, :, 0].reshape(ROWS, 128), p1[0, :, 1].reshape(ROWS, 128),
                   p1[1, :, 0].reshape(ROWS, 128), p1[1, :, 1].reshape(ROWS, 128),
                   W1a, b1a, W1b, b1b, W2a)

    gpad = jnp.stack([g0.reshape(-1), g1.reshape(-1)], axis=-1)
    p2 = _agg(gpad, srcp, dstp)
    o0, o1 = _mlp2(g0, g1,
                   p2[0, :, 0].reshape(ROWS, 128), p2[0, :, 1].reshape(ROWS, 128),
                   p2[1, :, 0].reshape(ROWS, 128), p2[1, :, 1].reshape(ROWS, 128),
                   b2a, W2b, b2b)
    return jnp.stack([o0.reshape(-1), o1.reshape(-1)], axis=-1)[:N]


# speedup vs baseline: 41.7221x; 41.7221x over previous
import functools

import jax
import jax.numpy as jnp
from jax import lax
from jax.experimental import pallas as pl
from jax.experimental.pallas import tpu as pltpu
from jax.experimental.pallas import tpu_sc as plsc

N = 100000
E = 3200000
NPAD = 100096
ROWS = NPAD // 128
NC = 2
NS = 16
NW = NC * NS
K = 8
RPW = 784
NBURST = RPW // K
EPAD = NW * RPW * 128
NSLICE = NPAD // NS
MW = 8


def _agg_body(tab_hbm, src_hbm, dst_hbm, out_hbm,
              tab_s, acc_s, src_v, dst_v, msg_v, gsem, ssem):
    cid = lax.axis_index("c")
    sid = lax.axis_index("s")
    wid = cid * NS + sid
    r0 = sid * NSLICE
    pltpu.sync_copy(tab_hbm.at[pl.ds(r0, NSLICE)], tab_s.at[pl.ds(r0, NSLICE)])
    pltpu.sync_copy(tab_hbm.at[pl.ds(r0, NSLICE)], acc_s.at[pl.ds(r0, NSLICE)])
    plsc.subcore_barrier()

    def burst(j, carry):
        base = j * K
        pltpu.sync_copy(src_hbm.at[wid, pl.ds(base, K)], src_v)
        pltpu.sync_copy(dst_hbm.at[wid, pl.ds(base, K)], dst_v)
        gds = [pltpu.async_copy(tab_s.at[src_v.at[b]], msg_v.at[b], gsem)
               for b in range(K)]
        for d in gds:
            d.wait()
        sds = [pltpu.async_copy(msg_v.at[b], acc_s.at[dst_v.at[b]], ssem,
                                add=True)
               for b in range(K)]
        for d in sds:
            d.wait()
        return carry

    lax.fori_loop(0, NBURST, burst, 0)
    plsc.subcore_barrier()
    pltpu.sync_copy(acc_s.at[pl.ds(r0, NSLICE)],
                    out_hbm.at[cid, pl.ds(r0, NSLICE)])


_agg = functools.partial(
    pl.kernel,
    out_type=jax.ShapeDtypeStruct((NC, NPAD, MW), jnp.float32),
    mesh=plsc.VectorSubcoreMesh(core_axis_name="c", subcore_axis_name="s",
                                num_cores=NC, num_subcores=NS),
    compiler_params=pltpu.CompilerParams(use_tc_tiling_on_sc=False),
    scratch_types=[
        pltpu.VMEM_SHARED((NPAD, MW), jnp.float32),
        pltpu.VMEM_SHARED((NPAD, MW), jnp.float32),
        pltpu.VMEM((K, 128), jnp.int32),
        pltpu.VMEM((K, 128), jnp.int32),
        pltpu.VMEM((K, 128, MW), jnp.float32),
        pltpu.SemaphoreType.DMA,
        pltpu.SemaphoreType.DMA,
    ],
)(_agg_body)


def _mlp1_body(x0, x1, a00, a01, a10, a11, w1a, b1a, w1b, b1b, w2a, g0, g1):
    z0 = a00[...] + a10[...] - x0[...]
    z1 = a01[...] + a11[...] - x1[...]
    u0 = jnp.maximum(z0 * w1a[0, 0] + z1 * w1a[1, 0] + b1a[0], 0.0)
    u1 = jnp.maximum(z0 * w1a[0, 1] + z1 * w1a[1, 1] + b1a[1], 0.0)
    acc0 = jnp.zeros_like(z0)
    acc1 = jnp.zeros_like(z0)
    for k in range(16):
        h = jnp.maximum(u0 * w1b[0, k] + u1 * w1b[1, k] + b1b[k], 0.0)
        acc0 = acc0 + h * w2a[k, 0]
        acc1 = acc1 + h * w2a[k, 1]
    g0[...] = acc0
    g1[...] = acc1


_VS = pl.BlockSpec(memory_space=pltpu.VMEM)
_SS = pl.BlockSpec(memory_space=pltpu.SMEM)

_mlp1 = pl.pallas_call(
    _mlp1_body,
    out_shape=[jax.ShapeDtypeStruct((ROWS, 128), jnp.float32)] * 2,
    in_specs=[_VS] * 6 + [_SS] * 5,
    out_specs=[_VS] * 2,
)


def _mlp2_body(g0, g1, c00, c01, c10, c11, b2a, w2b, b2b, o0, o1):
    s0 = jnp.maximum(c00[...] + c10[...] - g0[...] + b2a[0], 0.0)
    s1 = jnp.maximum(c01[...] + c11[...] - g1[...] + b2a[1], 0.0)
    h0 = s0 * w2b[0, 0] + s1 * w2b[1, 0] + b2b[0]
    h1 = s0 * w2b[0, 1] + s1 * w2b[1, 1] + b2b[1]
    m = jnp.maximum(h0, h1)
    lse = m + jnp.log(jnp.exp(h0 - m) + jnp.exp(h1 - m))
    o0[...] = h0 - lse
    o1[...] = h1 - lse


_mlp2 = pl.pallas_call(
    _mlp2_body,
    out_shape=[jax.ShapeDtypeStruct((ROWS, 128), jnp.float32)] * 2,
    in_specs=[_VS] * 6 + [_SS] * 3,
    out_specs=[_VS] * 2,
)


def kernel(x, edge_index, W1a, b1a, W1b, b1b, W2a, b2a, W2b, b2b):
    src = edge_index[0]
    dst = edge_index[1]
    padn = EPAD - E
    pad_idx = jnp.full((padn,), N, jnp.int32)
    srcp = jnp.concatenate([src, pad_idx]).reshape(NW, RPW, 128)
    dstp = jnp.concatenate([dst, pad_idx]).reshape(NW, RPW, 128)
    xpad = jnp.pad(x, ((0, NPAD - N), (0, MW - 2)))

    p1 = _agg(xpad, srcp, dstp)
    x0 = xpad[:, 0].reshape(ROWS, 128)
    x1 = xpad[:, 1].reshape(ROWS, 128)
    g0, g1 = _mlp1(x0, x1,
                   p1[0, :, 0].reshape(ROWS, 128), p1[0, :, 1].reshape(ROWS, 128),
                   p1[1, :, 0].reshape(ROWS, 128), p1[1, :, 1].reshape(ROWS, 128),
                   W1a, b1a, W1b, b1b, W2a)

    gflat = jnp.stack([g0.reshape(-1), g1.reshape(-1)], axis=-1)
    gpad = jnp.pad(gflat, ((0, 0), (0, MW - 2)))
    p2 = _agg(gpad, srcp, dstp)
    o0, o1 = _mlp2(g0, g1,
                   p2[0, :, 0].reshape(ROWS, 128), p2[0, :, 1].reshape(ROWS, 128),
                   p2[1, :, 0].reshape(ROWS, 128), p2[1, :, 1].reshape(ROWS, 128),
                   b2a, W2b, b2b)
    return jnp.stack([o0.reshape(-1), o1.reshape(-1)], axis=-1)[:N]


# speedup vs baseline: 54.2510x; 1.3003x over previous
import functools

import jax
import jax.numpy as jnp
from jax import lax
from jax.experimental import pallas as pl
from jax.experimental.pallas import tpu as pltpu
from jax.experimental.pallas import tpu_sc as plsc

N = 100000
E = 3200000
NPAD = 100096
ROWS = NPAD // 128
NC = 2
NS = 16
NW = NC * NS
K = 8
RPW = 784
NBURST = RPW // K
EPAD = NW * RPW * 128
NSLICE = NPAD // NS
MW = 8


NSLOT = 4
_DRAIN_ROWS = K * 128


def _agg_body(tab_hbm, src_hbm, dst_hbm, out_hbm,
              tab_s, acc_s, src_v, dst_v, msg_v, isem, gsem, ssem):
    cid = lax.axis_index("c")
    sid = lax.axis_index("s")
    wid = cid * NS + sid
    r0 = sid * NSLICE
    pltpu.sync_copy(tab_hbm.at[pl.ds(r0, NSLICE)], tab_s.at[pl.ds(r0, NSLICE)])
    pltpu.sync_copy(tab_hbm.at[pl.ds(r0, NSLICE)], acc_s.at[pl.ds(r0, NSLICE)])
    plsc.subcore_barrier()

    pltpu.sync_copy(src_hbm.at[wid, pl.ds(0, K)], src_v.at[0])
    pltpu.sync_copy(dst_hbm.at[wid, pl.ds(0, K)], dst_v.at[0])

    def burst(j, carry):
        s_now = lax.rem(j, NSLOT)
        s_nxt = lax.rem(j + 1, NSLOT)
        m_now = lax.rem(j, 2)

        @pl.when(j + 1 < NBURST)
        def _():
            pltpu.async_copy(src_hbm.at[wid, pl.ds((j + 1) * K, K)],
                             src_v.at[s_nxt], isem)
            pltpu.async_copy(dst_hbm.at[wid, pl.ds((j + 1) * K, K)],
                             dst_v.at[s_nxt], isem)

        @pl.when(j >= 2)
        def _():
            pltpu.make_async_copy(tab_hbm.at[pl.ds(0, _DRAIN_ROWS)],
                                  msg_v.at[0], ssem).wait()

        gds = [pltpu.async_copy(tab_s.at[src_v.at[s_now, b]],
                                msg_v.at[m_now, b], gsem)
               for b in range(K)]

        @pl.when(j + 1 < NBURST)
        def _():
            pltpu.make_async_copy(src_hbm.at[wid, pl.ds(0, K)],
                                  src_v.at[s_nxt], isem).wait()
            pltpu.make_async_copy(dst_hbm.at[wid, pl.ds(0, K)],
                                  dst_v.at[s_nxt], isem).wait()

        for d in gds:
            d.wait()
        for b in range(K):
            pltpu.async_copy(msg_v.at[m_now, b], acc_s.at[dst_v.at[s_now, b]],
                             ssem, add=True)
        return carry

    lax.fori_loop(0, NBURST, burst, 0)
    for _ in range(2):
        pltpu.make_async_copy(tab_hbm.at[pl.ds(0, _DRAIN_ROWS)],
                              msg_v.at[0], ssem).wait()
    plsc.subcore_barrier()
    pltpu.sync_copy(acc_s.at[pl.ds(r0, NSLICE)],
                    out_hbm.at[cid, pl.ds(r0, NSLICE)])


_agg = functools.partial(
    pl.kernel,
    out_type=jax.ShapeDtypeStruct((NC, NPAD, MW), jnp.float32),
    mesh=plsc.VectorSubcoreMesh(core_axis_name="c", subcore_axis_name="s",
                                num_cores=NC, num_subcores=NS),
    compiler_params=pltpu.CompilerParams(use_tc_tiling_on_sc=False),
    scratch_types=[
        pltpu.VMEM_SHARED((NPAD, MW), jnp.float32),
        pltpu.VMEM_SHARED((NPAD, MW), jnp.float32),
        pltpu.VMEM((NSLOT, K, 128), jnp.int32),
        pltpu.VMEM((NSLOT, K, 128), jnp.int32),
        pltpu.VMEM((2, K, 128, MW), jnp.float32),
        pltpu.SemaphoreType.DMA,
        pltpu.SemaphoreType.DMA,
        pltpu.SemaphoreType.DMA,
    ],
)(_agg_body)


def _mlp1_body(x0, x1, a00, a01, a10, a11, w1a, b1a, w1b, b1b, w2a, g0, g1):
    z0 = a00[...] + a10[...] - x0[...]
    z1 = a01[...] + a11[...] - x1[...]
    u0 = jnp.maximum(z0 * w1a[0, 0] + z1 * w1a[1, 0] + b1a[0], 0.0)
    u1 = jnp.maximum(z0 * w1a[0, 1] + z1 * w1a[1, 1] + b1a[1], 0.0)
    acc0 = jnp.zeros_like(z0)
    acc1 = jnp.zeros_like(z0)
    for k in range(16):
        h = jnp.maximum(u0 * w1b[0, k] + u1 * w1b[1, k] + b1b[k], 0.0)
        acc0 = acc0 + h * w2a[k, 0]
        acc1 = acc1 + h * w2a[k, 1]
    g0[...] = acc0
    g1[...] = acc1


_VS = pl.BlockSpec(memory_space=pltpu.VMEM)
_SS = pl.BlockSpec(memory_space=pltpu.SMEM)

_mlp1 = pl.pallas_call(
    _mlp1_body,
    out_shape=[jax.ShapeDtypeStruct((ROWS, 128), jnp.float32)] * 2,
    in_specs=[_VS] * 6 + [_SS] * 5,
    out_specs=[_VS] * 2,
)


def _mlp2_body(g0, g1, c00, c01, c10, c11, b2a, w2b, b2b, o0, o1):
    s0 = jnp.maximum(c00[...] + c10[...] - g0[...] + b2a[0], 0.0)
    s1 = jnp.maximum(c01[...] + c11[...] - g1[...] + b2a[1], 0.0)
    h0 = s0 * w2b[0, 0] + s1 * w2b[1, 0] + b2b[0]
    h1 = s0 * w2b[0, 1] + s1 * w2b[1, 1] + b2b[1]
    m = jnp.maximum(h0, h1)
    lse = m + jnp.log(jnp.exp(h0 - m) + jnp.exp(h1 - m))
    o0[...] = h0 - lse
    o1[...] = h1 - lse


_mlp2 = pl.pallas_call(
    _mlp2_body,
    out_shape=[jax.ShapeDtypeStruct((ROWS, 128), jnp.float32)] * 2,
    in_specs=[_VS] * 6 + [_SS] * 3,
    out_specs=[_VS] * 2,
)


def kernel(x, edge_index, W1a, b1a, W1b, b1b, W2a, b2a, W2b, b2b):
    src = edge_index[0]
    dst = edge_index[1]
    padn = EPAD - E
    pad_idx = jnp.full((padn,), N, jnp.int32)
    srcp = jnp.concatenate([src, pad_idx]).reshape(NW, RPW, 128)
    dstp = jnp.concatenate([dst, pad_idx]).reshape(NW, RPW, 128)
    xpad = jnp.pad(x, ((0, NPAD - N), (0, MW - 2)))

    p1 = _agg(xpad, srcp, dstp)
    x0 = xpad[:, 0].reshape(ROWS, 128)
    x1 = xpad[:, 1].reshape(ROWS, 128)
    g0, g1 = _mlp1(x0, x1,
                   p1[0, :, 0].reshape(ROWS, 128), p1[0, :, 1].reshape(ROWS, 128),
                   p1[1, :, 0].reshape(ROWS, 128), p1[1, :, 1].reshape(ROWS, 128),
                   W1a, b1a, W1b, b1b, W2a)

    gflat = jnp.stack([g0.reshape(-1), g1.reshape(-1)], axis=-1)
    gpad = jnp.pad(gflat, ((0, 0), (0, MW - 2)))
    p2 = _agg(gpad, srcp, dstp)
    o0, o1 = _mlp2(g0, g1,
                   p2[0, :, 0].reshape(ROWS, 128), p2[0, :, 1].reshape(ROWS, 128),
                   p2[1, :, 0].reshape(ROWS, 128), p2[1, :, 1].reshape(ROWS, 128),
                   b2a, W2b, b2b)
    return jnp.stack([o0.reshape(-1), o1.reshape(-1)], axis=-1)[:N]


# speedup vs baseline: 84.4121x; 1.5560x over previous
import functools

import jax
import jax.numpy as jnp
from jax import lax
from jax.experimental import pallas as pl
from jax.experimental.pallas import tpu as pltpu
from jax.experimental.pallas import tpu_sc as plsc

N = 100000
E = 3200000
NPAD = 100096
ROWS = NPAD // 128
NC = 2
NS = 16
NW = NC * NS
K = 8
ECH = E // 128
NBURST = 97
MW = 8
NSLICE = NPAD // NS
CH = 272
NCH = NSLICE // CH
CGRP = CH // 16


def _make_agg(nfeat):
    def body(*refs):
        (tabp_hbm, ev_hbm, out_hbm,
         tab_s, acc_s, src_v, dst_v, msg_v, vbuf, pvio,
         isem, psem, gsem, ssem) = refs
        cid = lax.axis_index("c")
        sid = lax.axis_index("s")
        wid = cid * NS + sid
        r0 = sid * NSLICE
        iota = lax.iota(jnp.int32, 16)
        zeros16 = iota * 0
        fzeros16 = jnp.zeros((16,), jnp.float32)

        def pro_load(c, b):
            for f in range(nfeat):
                pltpu.async_copy(tabp_hbm.at[f, pl.ds(r0 + c * CH, CH)],
                                 pvio.at[b, f], psem)

        def pv_drain(b):
            for f in range(nfeat):
                pltpu.make_async_copy(tabp_hbm.at[0, pl.ds(0, CH)],
                                      pvio.at[b, f], psem).wait()

        pro_load(0, 0)

        def stage(c, carry):
            b = lax.rem(c, 2)
            pv_drain(b)

            @pl.when(c + 1 < NCH)
            def _():
                pro_load(c + 1, 1 - b)

            def fill(i, cc):
                rows = i * 16 + iota
                for f in range(nfeat):
                    plsc.store_scatter(vbuf, [rows, zeros16 + f],
                                       pvio[b, f, pl.ds(i * 16, 16)])
                return cc
            lax.fori_loop(0, CGRP, fill, 0)
            pltpu.sync_copy(vbuf, tab_s.at[pl.ds(r0 + c * CH, CH)])

            @pl.when(cid == 1)
            def _():
                def zfill(i, cc):
                    rows = i * 16 + iota
                    for f in range(nfeat):
                        plsc.store_scatter(vbuf, [rows, zeros16 + f],
                                           fzeros16)
                    return cc
                lax.fori_loop(0, CGRP, zfill, 0)
            pltpu.sync_copy(vbuf, acc_s.at[pl.ds(r0 + c * CH, CH)])
            return carry
        lax.fori_loop(0, NCH, stage, 0)
        plsc.subcore_barrier()

        base = wid * 781 + jnp.minimum(wid, 8)

        def sdrain():
            pltpu.make_async_copy(ev_hbm.at[0, pl.ds(0, 8 * MW)],
                                  msg_v.at[0], ssem).wait()

        pltpu.sync_copy(ev_hbm.at[0, pl.ds(base, K)], src_v.at[0])
        pltpu.sync_copy(ev_hbm.at[1, pl.ds(base, K)], dst_v.at[0])

        def burst(j, carry):
            s_now = lax.rem(j, 2)
            s_nxt = lax.rem(j + 1, 2)
            m_now = lax.rem(j, 2)

            gds = [pltpu.async_copy(tab_s.at[src_v.at[s_now, b]],
                                    msg_v.at[m_now, b], gsem)
                   for b in range(K)]

            @pl.when(j >= 1)
            def _():
                sdrain()

            @pl.when(j + 1 < NBURST)
            def _():
                pltpu.async_copy(ev_hbm.at[0, pl.ds(base + (j + 1) * K, K)],
                                 src_v.at[s_nxt], isem)
                pltpu.async_copy(ev_hbm.at[1, pl.ds(base + (j + 1) * K, K)],
                                 dst_v.at[s_nxt], isem)

            for d in gds:
                d.wait()
            for b in range(K):
                pltpu.async_copy(msg_v.at[m_now, b],
                                 acc_s.at[dst_v.at[s_now, b]], ssem, add=True)

            @pl.when(j + 1 < NBURST)
            def _():
                pltpu.make_async_copy(ev_hbm.at[0, pl.ds(base, K)],
                                      src_v.at[s_nxt], isem).wait()
                pltpu.make_async_copy(ev_hbm.at[1, pl.ds(base, K)],
                                      dst_v.at[s_nxt], isem).wait()
            return carry

        lax.fori_loop(0, NBURST, burst, 0)
        sdrain()

        tbase = base + NBURST * K
        pltpu.sync_copy(ev_hbm.at[0, pl.ds(tbase, 5)],
                        src_v.at[0, pl.ds(0, 5)])
        pltpu.sync_copy(ev_hbm.at[1, pl.ds(tbase, 5)],
                        dst_v.at[0, pl.ds(0, 5)])

        @pl.when(wid < 8)
        def _():
            pltpu.sync_copy(ev_hbm.at[0, pl.ds(tbase + 5, 1)],
                            src_v.at[1, pl.ds(0, 1)])
            pltpu.sync_copy(ev_hbm.at[1, pl.ds(tbase + 5, 1)],
                            dst_v.at[1, pl.ds(0, 1)])

        tds = [pltpu.async_copy(tab_s.at[src_v.at[0, t]],
                                msg_v.at[0, t], gsem)
               for t in range(5)]
        for d in tds:
            d.wait()
        for t in range(5):
            pltpu.async_copy(msg_v.at[0, t], acc_s.at[dst_v.at[0, t]],
                             ssem, add=True).wait()

        @pl.when(wid < 8)
        def _():
            pltpu.async_copy(tab_s.at[src_v.at[1, 0]],
                             msg_v.at[0, 5], gsem).wait()
            pltpu.async_copy(msg_v.at[0, 5], acc_s.at[dst_v.at[1, 0]],
                             ssem, add=True).wait()

        plsc.subcore_barrier()

        def unstage(c, carry):
            b = lax.rem(c, 2)

            @pl.when(c >= 2)
            def _():
                pv_drain(b)

            pltpu.sync_copy(acc_s.at[pl.ds(r0 + c * CH, CH)], vbuf)

            def unfill(i, cc):
                rows = i * 16 + iota
                for f in range(nfeat):
                    v = plsc.load_gather(vbuf, [rows, zeros16 + f])
                    pvio[b, f, pl.ds(i * 16, 16)] = v
                return cc
            lax.fori_loop(0, CGRP, unfill, 0)
            for f in range(nfeat):
                pltpu.async_copy(pvio.at[b, f],
                                 out_hbm.at[cid, f, pl.ds(r0 + c * CH, CH)],
                                 psem)
            return carry
        lax.fori_loop(0, NCH, unstage, 0)
        pv_drain(0)
        pv_drain(1)

    return functools.partial(
        pl.kernel,
        out_type=jax.ShapeDtypeStruct((NC, nfeat, NPAD), jnp.float32),
        mesh=plsc.VectorSubcoreMesh(core_axis_name="c", subcore_axis_name="s",
                                    num_cores=NC, num_subcores=NS),
        compiler_params=pltpu.CompilerParams(use_tc_tiling_on_sc=False,
                                             needs_layout_passes=False),
        scratch_types=[
            pltpu.VMEM_SHARED((NPAD, MW), jnp.float32),
            pltpu.VMEM_SHARED((NPAD, MW), jnp.float32),
            pltpu.VMEM((2, K, 128), jnp.int32),
            pltpu.VMEM((2, K, 128), jnp.int32),
            pltpu.VMEM((2, K, 128, MW), jnp.float32),
            pltpu.VMEM((CH, MW), jnp.float32),
            pltpu.VMEM((2, nfeat, CH), jnp.float32),
            pltpu.SemaphoreType.DMA,
            pltpu.SemaphoreType.DMA,
            pltpu.SemaphoreType.DMA,
            pltpu.SemaphoreType.DMA,
        ],
    )(body)


_agg2 = _make_agg(2)
_agg8 = _make_agg(8)


def _b(v):
    return v.astype(jnp.bfloat16).astype(jnp.float32)


def _mlp1_body(a00, a01, a10, a11, w1a, b1a, w1b, b1b, ha, hb):
    z0 = _b(a00[...] + a10[...])
    z1 = _b(a01[...] + a11[...])
    u0 = _b(jnp.maximum(z0 * _b(w1a[0, 0]) + z1 * _b(w1a[1, 0]) + b1a[0], 0.0))
    u1 = _b(jnp.maximum(z0 * _b(w1a[0, 1]) + z1 * _b(w1a[1, 1]) + b1a[1], 0.0))
    for k in range(16):
        h = jnp.maximum(u0 * _b(w1b[0, k]) + u1 * _b(w1b[1, k]) + b1b[k], 0.0)
        if k < 8:
            ha[k] = h
        else:
            hb[k - 8] = h


_VS = pl.BlockSpec(memory_space=pltpu.VMEM)
_SS = pl.BlockSpec(memory_space=pltpu.SMEM)

_mlp1 = pl.pallas_call(
    _mlp1_body,
    out_shape=[jax.ShapeDtypeStruct((8, ROWS, 128), jnp.float32)] * 2,
    in_specs=[_VS] * 4 + [_SS] * 4,
    out_specs=[_VS] * 2,
)


def _mlp2_body(pa0, pa1, pb0, pb1, w2a, b2a, w2b, b2b, o0, o1):
    s0 = jnp.zeros((ROWS, 128), jnp.float32)
    s1 = jnp.zeros((ROWS, 128), jnp.float32)
    for k in range(16):
        if k < 8:
            z = _b(pa0[k] + pa1[k])
        else:
            z = _b(pb0[k - 8] + pb1[k - 8])
        s0 = s0 + z * _b(w2a[k, 0])
        s1 = s1 + z * _b(w2a[k, 1])
    r0 = _b(jnp.maximum(s0 + b2a[0], 0.0))
    r1 = _b(jnp.maximum(s1 + b2a[1], 0.0))
    h0 = r0 * _b(w2b[0, 0]) + r1 * _b(w2b[1, 0]) + b2b[0]
    h1 = r0 * _b(w2b[0, 1]) + r1 * _b(w2b[1, 1]) + b2b[1]
    m = jnp.maximum(h0, h1)
    lse = m + jnp.log(jnp.exp(h0 - m) + jnp.exp(h1 - m))
    o0[...] = h0 - lse
    o1[...] = h1 - lse


_mlp2 = pl.pallas_call(
    _mlp2_body,
    out_shape=[jax.ShapeDtypeStruct((ROWS, 128), jnp.float32)] * 2,
    in_specs=[_VS] * 4 + [_SS] * 4,
    out_specs=[_VS] * 2,
)


def kernel(x, edge_index, W1a, b1a, W1b, b1b, W2a, b2a, W2b, b2b):
    ev = edge_index.reshape(2, ECH, 128)
    xp = jnp.pad(x, ((0, NPAD - N), (0, 0))).T

    p1 = _agg2(xp, ev)
    ha, hb = _mlp1(p1[0, 0].reshape(ROWS, 128), p1[0, 1].reshape(ROWS, 128),
                   p1[1, 0].reshape(ROWS, 128), p1[1, 1].reshape(ROWS, 128),
                   W1a, b1a, W1b, b1b)

    p2a = _agg8(ha.reshape(8, NPAD), ev)
    p2b = _agg8(hb.reshape(8, NPAD), ev)
    o0, o1 = _mlp2(p2a[0].reshape(8, ROWS, 128), p2a[1].reshape(8, ROWS, 128),
                   p2b[0].reshape(8, ROWS, 128), p2b[1].reshape(8, ROWS, 128),
                   W2a, b2a, W2b, b2b)
    return jnp.stack([o0.reshape(-1), o1.reshape(-1)], axis=-1)[:N]
